# baseline (speedup 1.0000x reference)
import functools
import math

import jax
import jax.numpy as jnp
from jax import lax
from jax.experimental import pallas as pl
from jax.experimental.pallas import tpu as pltpu
from jax.experimental.pallas import tpu_sc as plsc

N = 16384
D = 64
L = 16
NW = 32
RPW = N // NW
C = 512
GB = 128


def _make_kernel(nnz_pad: int):
    nb = nnz_pad // C
    iters = max(1, math.ceil(math.log2(nb)))
    mesh = plsc.VectorSubcoreMesh(core_axis_name="c", subcore_axis_name="s")

    @functools.partial(
        pl.kernel,
        mesh=mesh,
        out_type=jax.ShapeDtypeStruct((N, D), jnp.float32),
        compiler_params=pltpu.CompilerParams(use_tc_tiling_on_sc=False),
        scratch_types=[
            pltpu.VMEM((L,), jnp.int32),
            pltpu.VMEM((C,), jnp.int32),
            pltpu.VMEM((C,), jnp.int32),
            pltpu.VMEM((C,), jnp.int32),
            pltpu.VMEM((C,), jnp.int32),
            pltpu.VMEM((C,), jnp.float32),
            pltpu.VMEM((C,), jnp.float32),
            pltpu.VMEM((C, D), jnp.float32),
            pltpu.VMEM((C, D), jnp.float32),
            pltpu.VMEM((RPW + 1, D), jnp.float32),
            pltpu.SemaphoreType.DMA,
            pltpu.SemaphoreType.DMA,
            pltpu.SemaphoreType.DMA,
            pltpu.SemaphoreType.DMA,
        ],
    )
    def spmm(values_hbm, weight_hbm, row_hbm, col_hbm, out_hbm,
             probe_v, col0, col1, row0, row1, val0, val1, rows0, rows1,
             acc_v, lsem0, lsem1, gsem0, gsem1):
        wid = lax.axis_index("s") * 2 + lax.axis_index("c")
        base = wid * RPW
        bufs = ((col0, row0, val0, rows0, lsem0, gsem0),
                (col1, row1, val1, rows1, lsem1, gsem1))

        def first_chunk_ge(target):
            def body(_, carry):
                lo, hi = carry
                mid = jnp.minimum((lo + hi) // 2, nb - 1)
                pltpu.sync_copy(row_hbm.at[pl.ds(mid * C, L)], probe_v)
                ge = probe_v[pl.ds(0, L)][0] >= target
                return (jnp.where(ge, lo, mid + 1), jnp.where(ge, mid, hi))
            lo, hi = lax.fori_loop(0, iters, body, (jnp.int32(0), jnp.int32(nb)))
            return hi

        j_lo = first_chunk_ge(base)
        j_hi = first_chunk_ge(base + RPW)
        j_start = jnp.maximum(j_lo - 1, 0)

        def linear_descs(j, b):
            col_v, row_v, val_v, _, lsem, _ = bufs[b]
            off = j * C
            return ((col_hbm.at[pl.ds(off, C)], col_v, lsem),
                    (row_hbm.at[pl.ds(off, C)], row_v, lsem),
                    (values_hbm.at[pl.ds(off, C)], val_v, lsem))

        def linear_start(j, b):
            for args in linear_descs(j, b):
                pltpu.async_copy(*args)

        def linear_wait(j, b):
            for args in linear_descs(j, b):
                pltpu.make_async_copy(*args).wait()

        def gather_descs(b):
            col_v, _, _, rows_v, _, gsem = bufs[b]
            return tuple(
                (weight_hbm.at[col_v.at[pl.ds(g * GB, GB)]],
                 rows_v.at[pl.ds(g * GB, GB)], gsem)
                for g in range(C // GB))

        def gather_start(b):
            for args in gather_descs(b):
                pltpu.async_copy(*args)

        def gather_wait(b):
            for args in gather_descs(b):
                pltpu.make_async_copy(*args).wait()

        def compute(b):
            _, row_v, val_v, rows_v, _, _ = bufs[b]

            def grp_body(g, _):
                gl = g * L
                rl = row_v[pl.ds(gl, L)] - base
                ok = (rl >= 0) & (rl < RPW)
                rr = jnp.where(ok, rl, RPW)
                vm = jnp.where(ok, val_v[pl.ds(gl, L)], 0.0)
                uniform = rl[0] == rl[L - 1]

                def fast(_):
                    r0 = rr[0]
                    nd = D // L
                    p = [jnp.zeros((L,), jnp.float32) for _ in range(nd)]
                    q = [jnp.zeros((L,), jnp.float32) for _ in range(nd)]
                    for j in range(L):
                        v_j = vm[j]
                        tgt = p if j % 2 == 0 else q
                        for d in range(nd):
                            tgt[d] = tgt[d] + v_j * rows_v[gl + j, pl.ds(d * L, L)]
                    for d in range(nd):
                        sl = pl.ds(d * L, L)
                        acc_v[r0, sl] = acc_v[r0, sl] + (p[d] + q[d])
                    return 0

                def slow(_):
                    for j in range(L):
                        r_j = rr[j]
                        v_j = vm[j]
                        for d in range(D // L):
                            sl = pl.ds(d * L, L)
                            acc_v[r_j, sl] = acc_v[r_j, sl] + v_j * rows_v[gl + j, sl]
                    return 0

                lax.cond(uniform, fast, slow, 0)
                return 0
            lax.fori_loop(0, C // L, grp_body, 0, unroll=2)

        linear_start(j_start, 0)

        zero = jnp.zeros((L,), jnp.float32)
        def zbody(r, _):
            for dcol in range(D // L):
                acc_v[r, pl.ds(dcol * L, L)] = zero
            return 0
        lax.fori_loop(0, RPW + 1, zbody, 0)

        linear_wait(j_start, 0)
        gather_start(0)
        linear_start(j_start + 1, 1)

        num = j_hi - j_start
        pairs = jnp.maximum(1, (num + 1) // 2)

        def pair_body(k, _):
            j0 = j_start + 2 * k
            linear_wait(j0 + 1, 1)
            gather_start(1)
            gather_wait(0)
            compute(0)
            linear_start(j0 + 2, 0)
            linear_wait(j0 + 2, 0)
            gather_start(0)
            gather_wait(1)
            compute(1)
            linear_start(j0 + 3, 1)
            return 0
        lax.fori_loop(0, pairs, pair_body, 0)

        gather_wait(0)
        linear_wait(j_start, 1)

        pltpu.sync_copy(acc_v.at[pl.ds(0, RPW)], out_hbm.at[pl.ds(base, RPW)])

    return spmm


def kernel(values, weight, row_idx, col_idx):
    nnz = values.shape[0]
    nnz_pad = (nnz // C + 6) * C
    pad = nnz_pad - nnz
    values_p = jnp.pad(values, (0, pad))
    row_p = jnp.pad(row_idx.astype(jnp.int32), (0, pad), constant_values=N)
    col_p = jnp.pad(col_idx.astype(jnp.int32), (0, pad))
    return _make_kernel(nnz_pad)(values_p, weight, row_p, col_p)


# speedup vs baseline: 1.3402x; 1.3402x over previous
import functools
import math

import jax
import jax.numpy as jnp
from jax import lax
from jax.experimental import pallas as pl
from jax.experimental.pallas import tpu as pltpu
from jax.experimental.pallas import tpu_sc as plsc

N = 16384
D = 64
L = 16
L2 = 32
NW = 32
RPW = N // NW
C = 1024
GB = 128


def _make_kernel(nnz_pad: int):
    nb = nnz_pad // C
    iters = max(1, math.ceil(math.log2(nb)))
    mesh = plsc.VectorSubcoreMesh(core_axis_name="c", subcore_axis_name="s")

    @functools.partial(
        pl.kernel,
        mesh=mesh,
        out_type=jax.ShapeDtypeStruct((N, D), jnp.float32),
        compiler_params=pltpu.CompilerParams(use_tc_tiling_on_sc=False, needs_layout_passes=False),
        scratch_types=[
            pltpu.VMEM((L,), jnp.int32),
            pltpu.VMEM((C,), jnp.int32),
            pltpu.VMEM((C,), jnp.int32),
            pltpu.VMEM((C,), jnp.int32),
            pltpu.VMEM((C,), jnp.int32),
            pltpu.VMEM((C,), jnp.float32),
            pltpu.VMEM((C,), jnp.float32),
            pltpu.VMEM((C, D), jnp.bfloat16),
            pltpu.VMEM((C, D), jnp.bfloat16),
            pltpu.VMEM((RPW + 1, D), jnp.float32),
            pltpu.SemaphoreType.DMA,
            pltpu.SemaphoreType.DMA,
            pltpu.SemaphoreType.DMA,
            pltpu.SemaphoreType.DMA,
        ],
    )
    def spmm(values_hbm, weight_hbm, row_hbm, col_hbm, out_hbm,
             probe_v, col0, col1, row0, row1, val0, val1, rows0, rows1,
             acc_v, lsem0, lsem1, gsem0, gsem1):
        wid = lax.axis_index("s") * 2 + lax.axis_index("c")
        base = wid * RPW
        bufs = ((col0, row0, val0, rows0, lsem0, gsem0),
                (col1, row1, val1, rows1, lsem1, gsem1))

        def first_chunk_ge(target):
            def body(_, carry):
                lo, hi = carry
                mid = jnp.minimum((lo + hi) // 2, nb - 1)
                pltpu.sync_copy(row_hbm.at[pl.ds(mid * C, L)], probe_v)
                ge = probe_v[pl.ds(0, L)][0] >= target
                return (jnp.where(ge, lo, mid + 1), jnp.where(ge, mid, hi))
            lo, hi = lax.fori_loop(0, iters, body, (jnp.int32(0), jnp.int32(nb)))
            return hi

        j_lo = first_chunk_ge(base)
        j_hi = first_chunk_ge(base + RPW)
        j_start = jnp.maximum(j_lo - 1, 0)

        def linear_descs(j, b):
            col_v, row_v, val_v, _, lsem, _ = bufs[b]
            off = j * C
            return ((col_hbm.at[pl.ds(off, C)], col_v, lsem),
                    (row_hbm.at[pl.ds(off, C)], row_v, lsem),
                    (values_hbm.at[pl.ds(off, C)], val_v, lsem))

        def linear_start(j, b):
            for args in linear_descs(j, b):
                pltpu.async_copy(*args)

        def linear_wait(j, b):
            for args in linear_descs(j, b):
                pltpu.make_async_copy(*args).wait()

        def gather_descs(b):
            col_v, _, _, rows_v, _, gsem = bufs[b]
            return tuple(
                (weight_hbm.at[col_v.at[pl.ds(g * GB, GB)]],
                 rows_v.at[pl.ds(g * GB, GB)], gsem)
                for g in range(C // GB))

        def gather_start(b):
            for args in gather_descs(b):
                pltpu.async_copy(*args)

        def gather_wait(b):
            for args in gather_descs(b):
                pltpu.make_async_copy(*args).wait()

        def compute(b):
            _, row_v, val_v, rows_v, _, _ = bufs[b]

            def grp_body(g, _):
                gl = g * L
                rl = row_v[pl.ds(gl, L)] - base
                ok = (rl >= 0) & (rl < RPW)
                rr = jnp.where(ok, rl, RPW)
                vm = jnp.where(ok, val_v[pl.ds(gl, L)], 0.0)
                uniform = rl[0] == rl[L - 1]

                def fast(_):
                    r0 = rr[0]
                    nd = D // L2
                    p = [jnp.zeros((L2,), jnp.bfloat16) for _ in range(nd)]
                    q = [jnp.zeros((L2,), jnp.bfloat16) for _ in range(nd)]
                    for j in range(L):
                        v16 = jnp.full((L,), vm[j], jnp.float32)
                        vb = plsc.pack(v16, v16,
                                       format=plsc.PackFormat.INTERLEAVED)
                        tgt = p if j % 2 == 0 else q
                        for d in range(nd):
                            tgt[d] = tgt[d] + vb * rows_v[gl + j, pl.ds(d * L2, L2)]
                    for d in range(nd):
                        pa, pb = plsc.unpack(p[d], format=plsc.PackFormat.INTERLEAVED)
                        qa, qb = plsc.unpack(q[d], format=plsc.PackFormat.INTERLEAVED)
                        sla = pl.ds(2 * d * L, L)
                        slb = pl.ds((2 * d + 1) * L, L)
                        acc_v[r0, sla] = acc_v[r0, sla] + (pa + qa)
                        acc_v[r0, slb] = acc_v[r0, slb] + (pb + qb)
                    return 0

                def slow(_):
                    for j in range(L):
                        r_j = rr[j]
                        v_j = vm[j]
                        for d in range(D // L2):
                            wa, wb = plsc.unpack(
                                rows_v[gl + j, pl.ds(d * L2, L2)],
                                format=plsc.PackFormat.INTERLEAVED)
                            sla = pl.ds(2 * d * L, L)
                            slb = pl.ds((2 * d + 1) * L, L)
                            acc_v[r_j, sla] = acc_v[r_j, sla] + v_j * wa
                            acc_v[r_j, slb] = acc_v[r_j, slb] + v_j * wb
                    return 0

                lax.cond(uniform, fast, slow, 0)
                return 0
            lax.fori_loop(0, C // L, grp_body, 0)

        linear_start(j_start, 0)

        zero = jnp.zeros((L,), jnp.float32)
        def zbody(r, _):
            for dcol in range(D // L):
                acc_v[r, pl.ds(dcol * L, L)] = zero
            return 0
        lax.fori_loop(0, RPW + 1, zbody, 0)

        linear_wait(j_start, 0)
        gather_start(0)
        linear_start(j_start + 1, 1)

        num = j_hi - j_start
        pairs = jnp.maximum(1, (num + 1) // 2)

        def pair_body(k, _):
            j0 = j_start + 2 * k
            linear_wait(j0 + 1, 1)
            gather_start(1)
            gather_wait(0)
            compute(0)
            linear_start(j0 + 2, 0)
            linear_wait(j0 + 2, 0)
            gather_start(0)
            gather_wait(1)
            compute(1)
            linear_start(j0 + 3, 1)
            return 0
        lax.fori_loop(0, pairs, pair_body, 0)

        gather_wait(0)
        linear_wait(j_start, 1)

        pltpu.sync_copy(acc_v.at[pl.ds(0, RPW)], out_hbm.at[pl.ds(base, RPW)])

    return spmm


def kernel(values, weight, row_idx, col_idx):
    nnz = values.shape[0]
    nnz_pad = (nnz // C + 6) * C
    pad = nnz_pad - nnz
    values_p = jnp.pad(values, (0, pad))
    row_p = jnp.pad(row_idx.astype(jnp.int32), (0, pad), constant_values=N)
    col_p = jnp.pad(col_idx.astype(jnp.int32), (0, pad))
    wb = weight.astype(jnp.bfloat16).reshape(N, D // L2, 2, L)
    wb = wb.transpose(0, 1, 3, 2).reshape(N, D)
    return _make_kernel(nnz_pad)(values_p, wb, row_p, col_p)


# speedup vs baseline: 1.5118x; 1.1281x over previous
import functools
import math

import jax
import jax.numpy as jnp
from jax import lax
from jax.experimental import pallas as pl
from jax.experimental.pallas import tpu as pltpu
from jax.experimental.pallas import tpu_sc as plsc

N = 16384
D = 64
L = 16
L2 = 32
NW = 32
RPW = N // NW
C = 1024
GB = 128


def _make_kernel(nnz_pad: int):
    nb = nnz_pad // C
    iters = max(1, math.ceil(math.log2(nb)))
    mesh = plsc.VectorSubcoreMesh(core_axis_name="c", subcore_axis_name="s")

    @functools.partial(
        pl.kernel,
        mesh=mesh,
        out_type=jax.ShapeDtypeStruct((N, D), jnp.float32),
        compiler_params=pltpu.CompilerParams(use_tc_tiling_on_sc=False, needs_layout_passes=False),
        scratch_types=[
            pltpu.VMEM((L,), jnp.int32),
            pltpu.VMEM((C,), jnp.int32),
            pltpu.VMEM((C,), jnp.int32),
            pltpu.VMEM((C + L,), jnp.int32),
            pltpu.VMEM((C + L,), jnp.int32),
            pltpu.VMEM((C,), jnp.float32),
            pltpu.VMEM((C,), jnp.float32),
            pltpu.VMEM((C, D), jnp.bfloat16),
            pltpu.VMEM((C, D), jnp.bfloat16),
            pltpu.VMEM((RPW + 1, D), jnp.float32),
            pltpu.SemaphoreType.DMA,
            pltpu.SemaphoreType.DMA,
            pltpu.SemaphoreType.DMA,
            pltpu.SemaphoreType.DMA,
        ],
    )
    def spmm(values_hbm, weight_hbm, row_hbm, col_hbm, out_hbm,
             probe_v, col0, col1, row0, row1, val0, val1, rows0, rows1,
             acc_v, lsem0, lsem1, gsem0, gsem1):
        wid = lax.axis_index("s") * 2 + lax.axis_index("c")
        base = wid * RPW
        bufs = ((col0, row0, val0, rows0, lsem0, gsem0),
                (col1, row1, val1, rows1, lsem1, gsem1))

        def first_chunk_ge(target):
            def body(_, carry):
                lo, hi = carry
                mid = jnp.minimum((lo + hi) // 2, nb - 1)
                pltpu.sync_copy(row_hbm.at[pl.ds(mid * C, L)], probe_v)
                ge = probe_v[pl.ds(0, L)][0] >= target
                return (jnp.where(ge, lo, mid + 1), jnp.where(ge, mid, hi))
            lo, hi = lax.fori_loop(0, iters, body, (jnp.int32(0), jnp.int32(nb)))
            return hi

        j_lo = first_chunk_ge(base)
        j_hi = first_chunk_ge(base + RPW)
        j_start = jnp.maximum(j_lo - 1, 0)

        def linear_descs(j, b):
            col_v, row_v, val_v, _, lsem, _ = bufs[b]
            off = j * C
            return ((col_hbm.at[pl.ds(off, C)], col_v, lsem),
                    (row_hbm.at[pl.ds(off, C + L)], row_v, lsem),
                    (values_hbm.at[pl.ds(off, C)], val_v, lsem))

        def linear_start(j, b):
            for args in linear_descs(j, b):
                pltpu.async_copy(*args)

        def linear_wait(j, b):
            for args in linear_descs(j, b):
                pltpu.make_async_copy(*args).wait()

        def gather_descs(b):
            col_v, _, _, rows_v, _, gsem = bufs[b]
            return tuple(
                (weight_hbm.at[col_v.at[pl.ds(g * GB, GB)]],
                 rows_v.at[pl.ds(g * GB, GB)], gsem)
                for g in range(C // GB))

        def gather_start(b):
            for args in gather_descs(b):
                pltpu.async_copy(*args)

        def gather_wait(b):
            for args in gather_descs(b):
                pltpu.make_async_copy(*args).wait()

        def compute(b):
            _, row_v, val_v, rows_v, _, _ = bufs[b]
            nd = D // L2

            def accumulate(r0, vmvec, gl):
                p = [jnp.zeros((L2,), jnp.bfloat16) for _ in range(nd)]
                q = [jnp.zeros((L2,), jnp.bfloat16) for _ in range(nd)]
                for j in range(L):
                    v16 = jnp.full((L,), vmvec[j], jnp.float32)
                    vb = plsc.pack(v16, v16, format=plsc.PackFormat.INTERLEAVED)
                    tgt = p if j % 2 == 0 else q
                    for d in range(nd):
                        tgt[d] = tgt[d] + vb * rows_v[gl + j, pl.ds(d * L2, L2)]
                for d in range(nd):
                    pa, pb = plsc.unpack(p[d], format=plsc.PackFormat.INTERLEAVED)
                    qa, qb = plsc.unpack(q[d], format=plsc.PackFormat.INTERLEAVED)
                    sla = pl.ds(2 * d * L, L)
                    slb = pl.ds((2 * d + 1) * L, L)
                    acc_v[r0, sla] = acc_v[r0, sla] + (pa + qa)
                    acc_v[r0, slb] = acc_v[r0, slb] + (pb + qb)

            def grp_body(g, _):
                gl = g * L
                rl = row_v[pl.ds(gl, L)] - base
                vraw = val_v[pl.ds(gl, L)]
                rl0 = rl[0]
                rl15 = rl[L - 1]
                uniform = rl0 == rl15

                def fast(_):
                    ok0 = (rl0 >= 0) & (rl0 < RPW)
                    r0 = jnp.where(ok0, rl0, RPW)
                    accumulate(r0, vraw, gl)
                    return 0

                def nonuniform(_):
                    rln = row_v[pl.ds(gl + 1, L)] - base
                    lane = lax.iota(jnp.int32, L)
                    m = (rl != rln) & (lane < L - 1)
                    nch = plsc.all_reduce_population_count(m)
                    two = nch[0] == 1

                    def dual(_):
                        bvec = plsc.all_reduce_ffs(m)
                        mask_a = lane <= bvec
                        vm_a = jnp.where(mask_a, vraw, 0.0)
                        vm_b = jnp.where(mask_a, 0.0, vraw)
                        ok_a = (rl0 >= 0) & (rl0 < RPW)
                        r_a = jnp.where(ok_a, rl0, RPW)
                        ok_b = (rl15 >= 0) & (rl15 < RPW)
                        r_b = jnp.where(ok_b, rl15, RPW)
                        accumulate(r_a, vm_a, gl)
                        accumulate(r_b, vm_b, gl)
                        return 0

                    def lanewise(_):
                        ok = (rl >= 0) & (rl < RPW)
                        rr = jnp.where(ok, rl, RPW)
                        vm = jnp.where(ok, vraw, 0.0)
                        for j in range(L):
                            r_j = rr[j]
                            v_j = vm[j]
                            for d in range(nd):
                                wa, wb = plsc.unpack(
                                    rows_v[gl + j, pl.ds(d * L2, L2)],
                                    format=plsc.PackFormat.INTERLEAVED)
                                sla = pl.ds(2 * d * L, L)
                                slb = pl.ds((2 * d + 1) * L, L)
                                acc_v[r_j, sla] = acc_v[r_j, sla] + v_j * wa
                                acc_v[r_j, slb] = acc_v[r_j, slb] + v_j * wb
                        return 0

                    lax.cond(two, dual, lanewise, 0)
                    return 0

                lax.cond(uniform, fast, nonuniform, 0)
                return 0
            lax.fori_loop(0, C // L, grp_body, 0)

        linear_start(j_start, 0)

        zero = jnp.zeros((L,), jnp.float32)
        def zbody(r, _):
            for dcol in range(D // L):
                acc_v[r, pl.ds(dcol * L, L)] = zero
            return 0
        lax.fori_loop(0, RPW + 1, zbody, 0)

        linear_wait(j_start, 0)
        gather_start(0)
        linear_start(j_start + 1, 1)

        num = j_hi - j_start
        pairs = jnp.maximum(1, (num + 1) // 2)

        def pair_body(k, _):
            j0 = j_start + 2 * k
            linear_wait(j0 + 1, 1)
            gather_start(1)
            gather_wait(0)
            compute(0)
            linear_start(j0 + 2, 0)
            linear_wait(j0 + 2, 0)
            gather_start(0)
            gather_wait(1)
            compute(1)
            linear_start(j0 + 3, 1)
            return 0
        lax.fori_loop(0, pairs, pair_body, 0)

        gather_wait(0)
        linear_wait(j_start, 1)

        pltpu.sync_copy(acc_v.at[pl.ds(0, RPW)], out_hbm.at[pl.ds(base, RPW)])

    return spmm


def kernel(values, weight, row_idx, col_idx):
    nnz = values.shape[0]
    nnz_pad = (nnz // C + 6) * C
    pad = nnz_pad - nnz
    values_p = jnp.pad(values, (0, pad))
    row_p = jnp.pad(row_idx.astype(jnp.int32), (0, pad), constant_values=N)
    col_p = jnp.pad(col_idx.astype(jnp.int32), (0, pad))
    wb = weight.astype(jnp.bfloat16).reshape(N, D // L2, 2, L)
    wb = wb.transpose(0, 1, 3, 2).reshape(N, D)
    return _make_kernel(nnz_pad)(values_p, wb, row_p, col_p)


# speedup vs baseline: 1.7343x; 1.1472x over previous
import functools
import math

import jax
import jax.numpy as jnp
from jax import lax
from jax.experimental import pallas as pl
from jax.experimental.pallas import tpu as pltpu
from jax.experimental.pallas import tpu_sc as plsc

N = 16384
D = 64
L = 16
L2 = 32
NW = 32
RPW = N // NW
C = 1024
GB = 128


def _make_kernel(nnz_pad: int):
    nb = nnz_pad // C
    iters = max(1, math.ceil(math.log2(nb)))
    mesh = plsc.VectorSubcoreMesh(core_axis_name="c", subcore_axis_name="s")

    @functools.partial(
        pl.kernel,
        mesh=mesh,
        out_type=jax.ShapeDtypeStruct((N, D), jnp.float32),
        compiler_params=pltpu.CompilerParams(use_tc_tiling_on_sc=False, needs_layout_passes=False),
        scratch_types=[
            pltpu.VMEM((L,), jnp.int32),
            pltpu.VMEM((C,), jnp.int32),
            pltpu.VMEM((C,), jnp.int32),
            pltpu.VMEM((C + L,), jnp.int32),
            pltpu.VMEM((C + L,), jnp.int32),
            pltpu.VMEM((C,), jnp.float32),
            pltpu.VMEM((C,), jnp.float32),
            pltpu.VMEM((C, D), jnp.bfloat16),
            pltpu.VMEM((C, D), jnp.bfloat16),
            pltpu.VMEM((RPW + 1, D), jnp.float32),
            pltpu.SemaphoreType.DMA,
            pltpu.SemaphoreType.DMA,
            pltpu.SemaphoreType.DMA,
            pltpu.SemaphoreType.DMA,
        ],
    )
    def spmm(values_hbm, weight_hbm, row_hbm, col_hbm, out_hbm,
             probe_v, col0, col1, row0, row1, val0, val1, rows0, rows1,
             acc_v, lsem0, lsem1, gsem0, gsem1):
        wid = lax.axis_index("s") * 2 + lax.axis_index("c")
        base = wid * RPW
        bufs = ((col0, row0, val0, rows0, lsem0, gsem0),
                (col1, row1, val1, rows1, lsem1, gsem1))

        def first_chunk_ge(target):
            def body(_, carry):
                lo, hi = carry
                mid = jnp.minimum((lo + hi) // 2, nb - 1)
                pltpu.sync_copy(row_hbm.at[pl.ds(mid * C, L)], probe_v)
                ge = probe_v[pl.ds(0, L)][0] >= target
                return (jnp.where(ge, lo, mid + 1), jnp.where(ge, mid, hi))
            lo, hi = lax.fori_loop(0, iters, body, (jnp.int32(0), jnp.int32(nb)))
            return hi

        j_lo = first_chunk_ge(base)
        j_hi = first_chunk_ge(base + RPW)
        j_start = jnp.maximum(j_lo - 1, 0)

        def linear_descs(j, b):
            col_v, row_v, val_v, _, lsem, _ = bufs[b]
            off = j * C
            return ((col_hbm.at[pl.ds(off, C)], col_v, lsem),
                    (row_hbm.at[pl.ds(off, C + L)], row_v, lsem),
                    (values_hbm.at[pl.ds(off, C)], val_v, lsem))

        def linear_start(j, b):
            for args in linear_descs(j, b):
                pltpu.async_copy(*args)

        def linear_wait(j, b):
            for args in linear_descs(j, b):
                pltpu.make_async_copy(*args).wait()

        def gather_descs(b):
            col_v, _, _, rows_v, _, gsem = bufs[b]
            return tuple(
                (weight_hbm.at[col_v.at[pl.ds(g * GB, GB)]],
                 rows_v.at[pl.ds(g * GB, GB)], gsem)
                for g in range(C // GB))

        def gather_start(b):
            for args in gather_descs(b):
                pltpu.async_copy(*args)

        def gather_wait(b):
            for args in gather_descs(b):
                pltpu.make_async_copy(*args).wait()

        def compute(b):
            _, row_v, val_v, rows_v, _, _ = bufs[b]
            nd = D // L2

            def accumulate(r0, vmvecs, gl):
                nch = 2 * len(vmvecs)
                ch = [[jnp.zeros((L2,), jnp.bfloat16) for _ in range(nd)]
                      for _ in range(nch)]
                for k, vmvec in enumerate(vmvecs):
                    for j in range(L):
                        v16 = jnp.full((L,), vmvec[j], jnp.float32)
                        vb = plsc.pack(v16, v16, format=plsc.PackFormat.INTERLEAVED)
                        tgt = ch[2 * k + (j % 2)]
                        for d in range(nd):
                            tgt[d] = tgt[d] + vb * rows_v[gl + k * L + j,
                                                          pl.ds(d * L2, L2)]
                for d in range(nd):
                    sa, sb = None, None
                    for c in ch:
                        ca, cb = plsc.unpack(c[d], format=plsc.PackFormat.INTERLEAVED)
                        sa = ca if sa is None else sa + ca
                        sb = cb if sb is None else sb + cb
                    sla = pl.ds(2 * d * L, L)
                    slb = pl.ds((2 * d + 1) * L, L)
                    acc_v[r0, sla] = acc_v[r0, sla] + sa
                    acc_v[r0, slb] = acc_v[r0, slb] + sb

            def group16(gl, rl, vraw, rl0, rl15):
                uniform = rl0 == rl15

                def fast(_):
                    ok0 = (rl0 >= 0) & (rl0 < RPW)
                    r0 = jnp.where(ok0, rl0, RPW)
                    accumulate(r0, [vraw], gl)
                    return 0

                def nonuniform(_):
                    rln = row_v[pl.ds(gl + 1, L)] - base
                    lane = lax.iota(jnp.int32, L)
                    m = (rl != rln) & (lane < L - 1)
                    nch = plsc.all_reduce_population_count(m)
                    two = nch[0] == 1

                    def dual(_):
                        bvec = plsc.all_reduce_ffs(m)
                        mask_a = lane <= bvec
                        vm_a = jnp.where(mask_a, vraw, 0.0)
                        vm_b = jnp.where(mask_a, 0.0, vraw)
                        ok_a = (rl0 >= 0) & (rl0 < RPW)
                        r_a = jnp.where(ok_a, rl0, RPW)
                        ok_b = (rl15 >= 0) & (rl15 < RPW)
                        r_b = jnp.where(ok_b, rl15, RPW)
                        accumulate(r_a, [vm_a], gl)
                        accumulate(r_b, [vm_b], gl)
                        return 0

                    def lanewise(_):
                        ok = (rl >= 0) & (rl < RPW)
                        rr = jnp.where(ok, rl, RPW)
                        vm = jnp.where(ok, vraw, 0.0)
                        for j in range(L):
                            r_j = rr[j]
                            v_j = vm[j]
                            for d in range(nd):
                                wa, wb = plsc.unpack(
                                    rows_v[gl + j, pl.ds(d * L2, L2)],
                                    format=plsc.PackFormat.INTERLEAVED)
                                sla = pl.ds(2 * d * L, L)
                                slb = pl.ds((2 * d + 1) * L, L)
                                acc_v[r_j, sla] = acc_v[r_j, sla] + v_j * wa
                                acc_v[r_j, slb] = acc_v[r_j, slb] + v_j * wb
                        return 0

                    lax.cond(two, dual, lanewise, 0)
                    return 0

                lax.cond(uniform, fast, nonuniform, 0)

            def grp_body(g, _):
                gl = g * (2 * L)
                rla = row_v[pl.ds(gl, L)] - base
                rlb = row_v[pl.ds(gl + L, L)] - base
                va = val_v[pl.ds(gl, L)]
                vb = val_v[pl.ds(gl + L, L)]
                r0 = rla[0]
                r31 = rlb[L - 1]

                def fast32(_):
                    ok0 = (r0 >= 0) & (r0 < RPW)
                    rr = jnp.where(ok0, r0, RPW)
                    accumulate(rr, [va, vb], gl)
                    return 0

                def split(_):
                    r15 = rla[L - 1]
                    r16 = rlb[0]
                    group16(gl, rla, va, r0, r15)
                    group16(gl + L, rlb, vb, r16, r31)
                    return 0

                lax.cond(r0 == r31, fast32, split, 0)
                return 0
            lax.fori_loop(0, C // (2 * L), grp_body, 0)

        linear_start(j_start, 0)

        zero = jnp.zeros((L,), jnp.float32)
        def zbody(r, _):
            for dcol in range(D // L):
                acc_v[r, pl.ds(dcol * L, L)] = zero
            return 0
        lax.fori_loop(0, RPW + 1, zbody, 0)

        linear_wait(j_start, 0)
        gather_start(0)
        linear_start(j_start + 1, 1)

        num = j_hi - j_start
        pairs = jnp.maximum(1, (num + 1) // 2)

        def pair_body(k, _):
            j0 = j_start + 2 * k
            linear_wait(j0 + 1, 1)
            gather_start(1)
            gather_wait(0)
            compute(0)
            linear_start(j0 + 2, 0)
            linear_wait(j0 + 2, 0)
            gather_start(0)
            gather_wait(1)
            compute(1)
            linear_start(j0 + 3, 1)
            return 0
        lax.fori_loop(0, pairs, pair_body, 0)

        gather_wait(0)
        linear_wait(j_start, 1)

        pltpu.sync_copy(acc_v.at[pl.ds(0, RPW)], out_hbm.at[pl.ds(base, RPW)])

    return spmm


def kernel(values, weight, row_idx, col_idx):
    nnz = values.shape[0]
    nnz_pad = (nnz // C + 6) * C
    pad = nnz_pad - nnz
    values_p = jnp.pad(values, (0, pad))
    row_p = jnp.pad(row_idx.astype(jnp.int32), (0, pad), constant_values=N)
    col_p = jnp.pad(col_idx.astype(jnp.int32), (0, pad))
    wb = weight.astype(jnp.bfloat16).reshape(N, D // L2, 2, L)
    wb = wb.transpose(0, 1, 3, 2).reshape(N, D)
    return _make_kernel(nnz_pad)(values_p, wb, row_p, col_p)
